# Initial kernel scaffold; baseline (speedup 1.0000x reference)
import jax
import jax.numpy as jnp
from jax.experimental import pallas as pl


def kernel(values, coords, new_coords, shift):
    raise NotImplementedError("write your pallas kernel here")


# speedup vs baseline: 13.9432x; 13.9432x over previous
import functools

import jax
import jax.numpy as jnp
from jax import lax
from jax.experimental import pallas as pl
from jax.experimental.pallas import tpu as pltpu
from jax.experimental.pallas import tpu_sc as plsc

N_IN = 2048
N_TOTAL = 8192
C = 128
K = 4


BQ = 512
_N_BLOCKS = N_TOTAL // BQ


def _topk_body(qx_ref, qy_ref, cx_ref, cy_ref, sx_ref, sy_ref, idx_ref):
    dx = (qx_ref[...] - sx_ref[...]) - cx_ref[...]
    dy = (qy_ref[...] - sy_ref[...]) - cy_ref[...]
    d = jnp.sqrt(dx * dx + dy * dy)
    jj = lax.broadcasted_iota(jnp.int32, (BQ, N_IN), 1)
    cols = []
    for _ in range(K):
        m = jnp.min(d, axis=1, keepdims=True)
        cand = jnp.where(d == m, jj, N_IN)
        amin = jnp.min(cand, axis=1, keepdims=True)
        cols.append(amin)
        d = jnp.where(jj == amin, jnp.float32(jnp.inf), d)
    idx_ref[...] = jnp.concatenate(cols, axis=1)


_topk = pl.pallas_call(
    _topk_body,
    grid=(_N_BLOCKS,),
    in_specs=[
        pl.BlockSpec((BQ, 1), lambda i: (i, 0)),
        pl.BlockSpec((BQ, 1), lambda i: (i, 0)),
        pl.BlockSpec((1, N_IN), lambda i: (0, 0)),
        pl.BlockSpec((1, N_IN), lambda i: (0, 0)),
        pl.BlockSpec((1, 1), lambda i: (0, 0)),
        pl.BlockSpec((1, 1), lambda i: (0, 0)),
    ],
    out_specs=pl.BlockSpec((BQ, K), lambda i: (i, 0)),
    out_shape=jax.ShapeDtypeStruct((N_TOTAL, K), jnp.int32),
)


_NC = 2
_NS = 16
_NW = _NC * _NS
_QPW = N_TOTAL // _NW
_GQ = 32
_NCHUNK = _QPW // _GQ


@functools.cache
def _make_sc_gather_mean():
    @functools.partial(
        pl.kernel,
        mesh=plsc.VectorSubcoreMesh(core_axis_name="c", subcore_axis_name="s"),
        out_type=jax.ShapeDtypeStruct((N_TOTAL, C), jnp.float32),
        scratch_types=[
            pltpu.VMEM((K * _GQ,), jnp.int32),
            pltpu.VMEM((K * _GQ, C), jnp.float32),
            pltpu.VMEM((_GQ, C), jnp.float32),
            pltpu.SemaphoreType.DMA,
        ],
    )
    def _sc_gather_mean(valt_hbm, idxf_hbm, out_hbm, idx_v, rows_v, out_v, sem):
        wid = lax.axis_index("s") * _NC + lax.axis_index("c")
        base_q = wid * _QPW

        def chunk(j, carry):
            q0 = base_q + j * _GQ
            pltpu.sync_copy(idxf_hbm.at[pl.ds(q0 * K, K * _GQ)], idx_v)
            pltpu.async_copy(valt_hbm.at[idx_v], rows_v, sem).wait()

            def one_q(q, c2):
                for c in range(C // 16):
                    s = pl.ds(c * 16, 16)
                    acc = rows_v[K * q, s] + rows_v[K * q + 1, s]
                    acc = acc + rows_v[K * q + 2, s]
                    acc = acc + rows_v[K * q + 3, s]
                    out_v[q, s] = acc * 0.25
                return c2

            lax.fori_loop(0, _GQ, one_q, 0)
            pltpu.sync_copy(out_v, out_hbm.at[pl.ds(q0, _GQ)])
            return carry

        lax.fori_loop(0, _NCHUNK, chunk, 0)

    return _sc_gather_mean


_BT = 512


def _tr_body(x_ref, y_ref):
    y_ref[...] = x_ref[...].T


_transpose = pl.pallas_call(
    _tr_body,
    grid=(N_TOTAL // _BT,),
    in_specs=[pl.BlockSpec((_BT, C), lambda i: (i, 0))],
    out_specs=pl.BlockSpec((C, _BT), lambda i: (0, i)),
    out_shape=jax.ShapeDtypeStruct((C, N_TOTAL), jnp.float32),
)


def kernel(values, coords, new_coords, shift):
    all_coords = jnp.concatenate([coords, new_coords], axis=0)
    qx = all_coords[:, 0:1]
    qy = all_coords[:, 1:2]
    cx = coords[:, 0].reshape(1, N_IN)
    cy = coords[:, 1].reshape(1, N_IN)
    sx = shift[0].reshape(1, 1)
    sy = shift[1].reshape(1, 1)
    idx = _topk(qx, qy, cx, cy, sx, sy)
    valt = values.T
    out_t = _make_sc_gather_mean()(valt, idx.reshape(-1))
    return _transpose(out_t)


# speedup vs baseline: 14.9851x; 1.0747x over previous
import functools

import jax
import jax.numpy as jnp
from jax import lax
from jax.experimental import pallas as pl
from jax.experimental.pallas import tpu as pltpu
from jax.experimental.pallas import tpu_sc as plsc

N_IN = 2048
N_TOTAL = 8192
C = 128
K = 4


BQ = 512
_N_BLOCKS = N_TOTAL // BQ


def _topk_body(qx_ref, qy_ref, cx_ref, cy_ref, sx_ref, sy_ref, idx_ref):
    dx = (qx_ref[...] - sx_ref[...]) - cx_ref[...]
    dy = (qy_ref[...] - sy_ref[...]) - cy_ref[...]
    d = jnp.sqrt(dx * dx + dy * dy)
    jj = lax.broadcasted_iota(jnp.int32, (BQ, N_IN), 1)
    cols = []
    for _ in range(K):
        m = jnp.min(d, axis=1, keepdims=True)
        cand = jnp.where(d == m, jj, N_IN)
        amin = jnp.min(cand, axis=1, keepdims=True)
        cols.append(amin)
        d = jnp.where(jj == amin, jnp.float32(jnp.inf), d)
    idx_ref[...] = jnp.concatenate(cols, axis=1)


_topk = pl.pallas_call(
    _topk_body,
    grid=(_N_BLOCKS,),
    in_specs=[
        pl.BlockSpec((BQ, 1), lambda i: (i, 0)),
        pl.BlockSpec((BQ, 1), lambda i: (i, 0)),
        pl.BlockSpec((1, N_IN), lambda i: (0, 0)),
        pl.BlockSpec((1, N_IN), lambda i: (0, 0)),
        pl.BlockSpec((1, 1), lambda i: (0, 0)),
        pl.BlockSpec((1, 1), lambda i: (0, 0)),
    ],
    out_specs=pl.BlockSpec((BQ, K), lambda i: (i, 0)),
    out_shape=jax.ShapeDtypeStruct((N_TOTAL, K), jnp.int32),
)


_NC = 2
_NS = 16
_NW = _NC * _NS
_QPW = N_TOTAL // _NW
_GQ = 32
_NCHUNK = _QPW // _GQ


@functools.cache
def _make_sc_gather_mean():
    @functools.partial(
        pl.kernel,
        mesh=plsc.VectorSubcoreMesh(core_axis_name="c", subcore_axis_name="s"),
        out_type=jax.ShapeDtypeStruct((N_TOTAL, C), jnp.float32),
        scratch_types=[
            pltpu.VMEM((_NCHUNK, K * _GQ), jnp.int32),
            pltpu.VMEM((K * _GQ, C), jnp.float32),
            pltpu.VMEM((K * _GQ, C), jnp.float32),
            pltpu.VMEM((_QPW, C), jnp.float32),
            pltpu.SemaphoreType.DMA,
            pltpu.SemaphoreType.DMA,
        ],
    )
    def _sc_gather_mean(valt_hbm, idx3_hbm, out_hbm, idx_v, rows0, rows1,
                        out_v, sem0, sem1):
        wid = lax.axis_index("s") * _NC + lax.axis_index("c")
        base_q = wid * _QPW
        rows = (rows0, rows1)
        sems = (sem0, sem1)

        pltpu.sync_copy(idx3_hbm.at[wid], idx_v)
        cp = pltpu.async_copy(valt_hbm.at[idx_v.at[0]], rows0, sem0)
        pending = [cp]

        for j in range(_NCHUNK):
            if j + 1 < _NCHUNK:
                nxt = (j + 1) % 2
                pending.append(pltpu.async_copy(
                    valt_hbm.at[idx_v.at[j + 1]], rows[nxt], sems[nxt]))
            pending.pop(0).wait()
            rv = rows[j % 2]
            q0 = j * _GQ

            def one_q(q, carry, rv=rv, q0=q0):
                for c in range(C // 16):
                    s = pl.ds(c * 16, 16)
                    acc = rv[K * q, s] + rv[K * q + 1, s]
                    acc = acc + rv[K * q + 2, s]
                    acc = acc + rv[K * q + 3, s]
                    out_v[q0 + q, s] = acc * 0.25
                return carry

            lax.fori_loop(0, _GQ, one_q, 0)

        pltpu.sync_copy(out_v, out_hbm.at[pl.ds(base_q, _QPW)])

    return _sc_gather_mean


_BT = 512


def _tr_body(x_ref, y_ref):
    y_ref[...] = x_ref[...].T


_transpose = pl.pallas_call(
    _tr_body,
    grid=(N_TOTAL // _BT,),
    in_specs=[pl.BlockSpec((_BT, C), lambda i: (i, 0))],
    out_specs=pl.BlockSpec((C, _BT), lambda i: (0, i)),
    out_shape=jax.ShapeDtypeStruct((C, N_TOTAL), jnp.float32),
)


def kernel(values, coords, new_coords, shift):
    all_coords = jnp.concatenate([coords, new_coords], axis=0)
    qx = all_coords[:, 0:1]
    qy = all_coords[:, 1:2]
    cx = coords[:, 0].reshape(1, N_IN)
    cy = coords[:, 1].reshape(1, N_IN)
    sx = shift[0].reshape(1, 1)
    sy = shift[1].reshape(1, 1)
    idx = _topk(qx, qy, cx, cy, sx, sy)
    valt = values.T
    idx3 = idx.reshape(_NW, _NCHUNK, K * _GQ)
    out_t = _make_sc_gather_mean()(valt, idx3)
    return _transpose(out_t)


# speedup vs baseline: 18.0601x; 1.2052x over previous
import functools

import jax
import jax.numpy as jnp
from jax import lax
from jax.experimental import pallas as pl
from jax.experimental.pallas import tpu as pltpu
from jax.experimental.pallas import tpu_sc as plsc

N_IN = 2048
N_TOTAL = 8192
C = 128
K = 4


BQ = 512
_N_BLOCKS = N_TOTAL // BQ


def _topk_body(qx_ref, qy_ref, cx_ref, cy_ref, sx_ref, sy_ref, idx_ref):
    dx = (qx_ref[...] - sx_ref[...]) - cx_ref[...]
    dy = (qy_ref[...] - sy_ref[...]) - cy_ref[...]
    d = dx * dx + dy * dy
    jj = lax.broadcasted_iota(jnp.int32, (BQ, N_IN), 1).astype(jnp.float32)
    cols = []
    for _ in range(K):
        m = jnp.min(d, axis=1, keepdims=True)
        cand = jnp.where(d == m, jj, jnp.float32(N_IN))
        amin = jnp.min(cand, axis=1, keepdims=True)
        cols.append(amin)
        d = jnp.where(jj == amin, jnp.float32(jnp.inf), d)
    idx_ref[...] = jnp.concatenate(cols, axis=1).astype(jnp.int32)


_topk = pl.pallas_call(
    _topk_body,
    grid=(_N_BLOCKS,),
    in_specs=[
        pl.BlockSpec((BQ, 1), lambda i: (i, 0)),
        pl.BlockSpec((BQ, 1), lambda i: (i, 0)),
        pl.BlockSpec((1, N_IN), lambda i: (0, 0)),
        pl.BlockSpec((1, N_IN), lambda i: (0, 0)),
        pl.BlockSpec((1, 1), lambda i: (0, 0)),
        pl.BlockSpec((1, 1), lambda i: (0, 0)),
    ],
    out_specs=pl.BlockSpec((BQ, K), lambda i: (i, 0)),
    out_shape=jax.ShapeDtypeStruct((N_TOTAL, K), jnp.int32),
)


_NC = 2
_NS = 16
_NW = _NC * _NS
_QPW = N_TOTAL // _NW
_GQ = 32
_NCHUNK = _QPW // _GQ


@functools.cache
def _make_sc_gather_mean():
    @functools.partial(
        pl.kernel,
        mesh=plsc.VectorSubcoreMesh(core_axis_name="c", subcore_axis_name="s"),
        out_type=jax.ShapeDtypeStruct((N_TOTAL, C), jnp.float32),
        scratch_types=[
            pltpu.VMEM((_NCHUNK, K * _GQ), jnp.int32),
            pltpu.VMEM((K * _GQ, C), jnp.float32),
            pltpu.VMEM((K * _GQ, C), jnp.float32),
            pltpu.VMEM((_QPW, C), jnp.float32),
            pltpu.SemaphoreType.DMA,
            pltpu.SemaphoreType.DMA,
        ],
    )
    def _sc_gather_mean(valt_hbm, idx3_hbm, out_hbm, idx_v, rows0, rows1,
                        out_v, sem0, sem1):
        wid = lax.axis_index("s") * _NC + lax.axis_index("c")
        base_q = wid * _QPW
        rows = (rows0, rows1)
        sems = (sem0, sem1)

        pltpu.sync_copy(idx3_hbm.at[wid], idx_v)
        cp = pltpu.async_copy(valt_hbm.at[idx_v.at[0]], rows0, sem0)
        pending = [cp]

        for j in range(_NCHUNK):
            if j + 1 < _NCHUNK:
                nxt = (j + 1) % 2
                pending.append(pltpu.async_copy(
                    valt_hbm.at[idx_v.at[j + 1]], rows[nxt], sems[nxt]))
            pending.pop(0).wait()
            rv = rows[j % 2]
            q0 = j * _GQ

            def one_q(q, carry, rv=rv, q0=q0):
                for c in range(C // 16):
                    s = pl.ds(c * 16, 16)
                    acc = rv[K * q, s] + rv[K * q + 1, s]
                    acc = acc + rv[K * q + 2, s]
                    acc = acc + rv[K * q + 3, s]
                    out_v[q0 + q, s] = acc * 0.25
                return carry

            lax.fori_loop(0, _GQ, one_q, 0)

        pltpu.sync_copy(out_v, out_hbm.at[pl.ds(base_q, _QPW)])

    return _sc_gather_mean


_BT = 512


def _tr_body(x_ref, y_ref):
    y_ref[...] = x_ref[...].T


_transpose = pl.pallas_call(
    _tr_body,
    grid=(N_TOTAL // _BT,),
    in_specs=[pl.BlockSpec((_BT, C), lambda i: (i, 0))],
    out_specs=pl.BlockSpec((C, _BT), lambda i: (0, i)),
    out_shape=jax.ShapeDtypeStruct((C, N_TOTAL), jnp.float32),
)


def kernel(values, coords, new_coords, shift):
    all_coords = jnp.concatenate([coords, new_coords], axis=0)
    qx = all_coords[:, 0:1]
    qy = all_coords[:, 1:2]
    cx = coords[:, 0].reshape(1, N_IN)
    cy = coords[:, 1].reshape(1, N_IN)
    sx = shift[0].reshape(1, 1)
    sy = shift[1].reshape(1, 1)
    idx = _topk(qx, qy, cx, cy, sx, sy)
    valt = values.T
    idx3 = idx.reshape(_NW, _NCHUNK, K * _GQ)
    out_t = _make_sc_gather_mean()(valt, idx3)
    return _transpose(out_t)


# speedup vs baseline: 18.2195x; 1.0088x over previous
import functools

import jax
import jax.numpy as jnp
from jax import lax
from jax.experimental import pallas as pl
from jax.experimental.pallas import tpu as pltpu
from jax.experimental.pallas import tpu_sc as plsc

N_IN = 2048
N_TOTAL = 8192
C = 128
K = 4


BQ = 512
_N_BLOCKS = N_TOTAL // BQ


def _topk_body(cq_ref, nq_ref, cf_ref, sh_ref, v_ref, idx_ref, valt_ref):
    i = pl.program_id(0)
    qblk = jnp.where(i < N_IN // BQ, cq_ref[...], nq_ref[...])
    ct = cf_ref[...].T
    s = sh_ref[...]
    dx = (qblk[:, 0:1] - s[0:1, 0:1]) - ct[0:1, :]
    dy = (qblk[:, 1:2] - s[0:1, 1:2]) - ct[1:2, :]
    d = dx * dx + dy * dy
    jj = lax.broadcasted_iota(jnp.int32, (BQ, N_IN), 1).astype(jnp.float32)
    cols = []
    for k in range(K):
        m = jnp.min(d, axis=1, keepdims=True)
        cand = jnp.where(d == m, jj, jnp.float32(N_IN))
        amin = jnp.min(cand, axis=1, keepdims=True)
        cols.append(amin)
        if k + 1 < K:
            d = jnp.where(jj == amin, jnp.float32(jnp.inf), d)
    idx_ref[...] = jnp.concatenate(cols, axis=1).astype(jnp.int32)
    valt_ref[...] = v_ref[...].T


_topk = pl.pallas_call(
    _topk_body,
    grid=(_N_BLOCKS,),
    in_specs=[
        pl.BlockSpec((BQ, 2), lambda i: (jnp.minimum(i, N_IN // BQ - 1), 0)),
        pl.BlockSpec((BQ, 2), lambda i: (jnp.maximum(i - N_IN // BQ, 0), 0)),
        pl.BlockSpec((N_IN, 2), lambda i: (0, 0)),
        pl.BlockSpec((1, 2), lambda i: (0, 0)),
        pl.BlockSpec((C, N_IN // _N_BLOCKS), lambda i: (0, i)),
    ],
    out_specs=(
        pl.BlockSpec((BQ, K), lambda i: (i, 0)),
        pl.BlockSpec((N_IN // _N_BLOCKS, C), lambda i: (i, 0)),
    ),
    out_shape=(
        jax.ShapeDtypeStruct((N_TOTAL, K), jnp.int32),
        jax.ShapeDtypeStruct((N_IN, C), jnp.float32),
    ),
)


_NC = 2
_NS = 16
_NW = _NC * _NS
_QPW = N_TOTAL // _NW
_GQ = 32
_NCHUNK = _QPW // _GQ


@functools.cache
def _make_sc_gather_mean():
    @functools.partial(
        pl.kernel,
        mesh=plsc.VectorSubcoreMesh(core_axis_name="c", subcore_axis_name="s"),
        out_type=jax.ShapeDtypeStruct((N_TOTAL, C), jnp.float32),
        scratch_types=[
            pltpu.VMEM((_NCHUNK, K * _GQ), jnp.int32),
            pltpu.VMEM((K * _GQ, C), jnp.float32),
            pltpu.VMEM((K * _GQ, C), jnp.float32),
            pltpu.VMEM((_QPW, C), jnp.float32),
            pltpu.SemaphoreType.DMA,
            pltpu.SemaphoreType.DMA,
        ],
    )
    def _sc_gather_mean(valt_hbm, idx3_hbm, out_hbm, idx_v, rows0, rows1,
                        out_v, sem0, sem1):
        wid = lax.axis_index("s") * _NC + lax.axis_index("c")
        base_q = wid * _QPW
        rows = (rows0, rows1)
        sems = (sem0, sem1)

        pltpu.sync_copy(idx3_hbm.at[wid], idx_v)
        cp = pltpu.async_copy(valt_hbm.at[idx_v.at[0]], rows0, sem0)
        pending = [cp]

        for j in range(_NCHUNK):
            if j + 1 < _NCHUNK:
                nxt = (j + 1) % 2
                pending.append(pltpu.async_copy(
                    valt_hbm.at[idx_v.at[j + 1]], rows[nxt], sems[nxt]))
            pending.pop(0).wait()
            rv = rows[j % 2]
            q0 = j * _GQ

            def one_q(q, carry, rv=rv, q0=q0):
                for c in range(C // 16):
                    s = pl.ds(c * 16, 16)
                    acc = rv[K * q, s] + rv[K * q + 1, s]
                    acc = acc + rv[K * q + 2, s]
                    acc = acc + rv[K * q + 3, s]
                    out_v[q0 + q, s] = acc * 0.25
                return carry

            lax.fori_loop(0, _GQ, one_q, 0)

        pltpu.sync_copy(out_v, out_hbm.at[pl.ds(base_q, _QPW)])

    return _sc_gather_mean


_BT = 512


def _tr_body(x_ref, y_ref):
    y_ref[...] = x_ref[...].T


_transpose = pl.pallas_call(
    _tr_body,
    grid=(N_TOTAL // _BT,),
    in_specs=[pl.BlockSpec((_BT, C), lambda i: (i, 0))],
    out_specs=pl.BlockSpec((C, _BT), lambda i: (0, i)),
    out_shape=jax.ShapeDtypeStruct((C, N_TOTAL), jnp.float32),
)


def kernel(values, coords, new_coords, shift):
    idx, valt = _topk(coords, new_coords, coords, shift.reshape(1, 2), values)
    idx3 = idx.reshape(_NW, _NCHUNK, K * _GQ)
    out_t = _make_sc_gather_mean()(valt, idx3)
    return _transpose(out_t)


# speedup vs baseline: 19.5146x; 1.0711x over previous
import functools

import jax
import jax.numpy as jnp
from jax import lax
from jax.experimental import pallas as pl
from jax.experimental.pallas import tpu as pltpu
from jax.experimental.pallas import tpu_sc as plsc

N_IN = 2048
N_TOTAL = 8192
C = 128
K = 4


BQ = 1024
_N_BLOCKS = N_TOTAL // BQ


def _topk_body(cq_ref, nq_ref, cf_ref, sh_ref, v_ref, idx_ref, valt_ref):
    i = pl.program_id(0)
    qblk = jnp.where(i < N_IN // BQ, cq_ref[...], nq_ref[...])
    ct = cf_ref[...].T
    s = sh_ref[...]
    dx = (qblk[:, 0:1] - s[0:1, 0:1]) - ct[0:1, :]
    dy = (qblk[:, 1:2] - s[0:1, 1:2]) - ct[1:2, :]
    d = dx * dx + dy * dy
    jj = lax.broadcasted_iota(jnp.int32, (BQ, N_IN), 1).astype(jnp.float32)
    cols = []
    for k in range(K):
        m = jnp.min(d, axis=1, keepdims=True)
        cand = jnp.where(d == m, jj, jnp.float32(N_IN))
        amin = jnp.min(cand, axis=1, keepdims=True)
        cols.append(amin)
        if k + 1 < K:
            d = jnp.where(jj == amin, jnp.float32(jnp.inf), d)
    idx_ref[...] = jnp.concatenate(cols, axis=1).astype(jnp.int32)
    valt_ref[...] = v_ref[...].T


_topk = pl.pallas_call(
    _topk_body,
    grid=(_N_BLOCKS,),
    in_specs=[
        pl.BlockSpec((BQ, 2), lambda i: (jnp.minimum(i, N_IN // BQ - 1), 0)),
        pl.BlockSpec((BQ, 2), lambda i: (jnp.maximum(i - N_IN // BQ, 0), 0)),
        pl.BlockSpec((N_IN, 2), lambda i: (0, 0)),
        pl.BlockSpec((1, 2), lambda i: (0, 0)),
        pl.BlockSpec((C, N_IN // _N_BLOCKS), lambda i: (0, i)),
    ],
    out_specs=(
        pl.BlockSpec((BQ, K), lambda i: (i, 0)),
        pl.BlockSpec((N_IN // _N_BLOCKS, C), lambda i: (i, 0)),
    ),
    out_shape=(
        jax.ShapeDtypeStruct((N_TOTAL, K), jnp.int32),
        jax.ShapeDtypeStruct((N_IN, C), jnp.float32),
    ),
)


_NC = 2
_NS = 16
_NW = _NC * _NS
_QPW = N_TOTAL // _NW
_GQ = 32
_NCHUNK = _QPW // _GQ


@functools.cache
def _make_sc_gather_mean():
    @functools.partial(
        pl.kernel,
        mesh=plsc.VectorSubcoreMesh(core_axis_name="c", subcore_axis_name="s"),
        out_type=jax.ShapeDtypeStruct((N_TOTAL, C), jnp.float32),
        scratch_types=[
            pltpu.VMEM((_NCHUNK, K * _GQ), jnp.int32),
            pltpu.VMEM((K * _GQ, C), jnp.float32),
            pltpu.VMEM((K * _GQ, C), jnp.float32),
            pltpu.VMEM((_QPW, C), jnp.float32),
            pltpu.SemaphoreType.DMA,
            pltpu.SemaphoreType.DMA,
        ],
    )
    def _sc_gather_mean(valt_hbm, idx3_hbm, out_hbm, idx_v, rows0, rows1,
                        out_v, sem0, sem1):
        wid = lax.axis_index("s") * _NC + lax.axis_index("c")
        base_q = wid * _QPW
        rows = (rows0, rows1)
        sems = (sem0, sem1)

        pltpu.sync_copy(idx3_hbm.at[wid], idx_v)
        cp = pltpu.async_copy(valt_hbm.at[idx_v.at[0]], rows0, sem0)
        pending = [cp]

        for j in range(_NCHUNK):
            if j + 1 < _NCHUNK:
                nxt = (j + 1) % 2
                pending.append(pltpu.async_copy(
                    valt_hbm.at[idx_v.at[j + 1]], rows[nxt], sems[nxt]))
            pending.pop(0).wait()
            rv = rows[j % 2]
            q0 = j * _GQ

            def one_q(q, carry, rv=rv, q0=q0):
                for c in range(C // 16):
                    s = pl.ds(c * 16, 16)
                    acc = rv[K * q, s] + rv[K * q + 1, s]
                    acc = acc + rv[K * q + 2, s]
                    acc = acc + rv[K * q + 3, s]
                    out_v[q0 + q, s] = acc * 0.25
                return carry

            lax.fori_loop(0, _GQ, one_q, 0)

        pltpu.sync_copy(out_v, out_hbm.at[pl.ds(base_q, _QPW)])

    return _sc_gather_mean


_BT = 2048


def _tr_body(x_ref, y_ref):
    y_ref[...] = x_ref[...].T


_transpose = pl.pallas_call(
    _tr_body,
    grid=(N_TOTAL // _BT,),
    in_specs=[pl.BlockSpec((_BT, C), lambda i: (i, 0))],
    out_specs=pl.BlockSpec((C, _BT), lambda i: (0, i)),
    out_shape=jax.ShapeDtypeStruct((C, N_TOTAL), jnp.float32),
)


def kernel(values, coords, new_coords, shift):
    idx, valt = _topk(coords, new_coords, coords, shift.reshape(1, 2), values)
    idx3 = idx.reshape(_NW, _NCHUNK, K * _GQ)
    out_t = _make_sc_gather_mean()(valt, idx3)
    return _transpose(out_t)


# speedup vs baseline: 21.5418x; 1.1039x over previous
import functools

import jax
import jax.numpy as jnp
from jax import lax
from jax.experimental import pallas as pl
from jax.experimental.pallas import tpu as pltpu
from jax.experimental.pallas import tpu_sc as plsc

N_IN = 2048
N_TOTAL = 8192
C = 128
K = 4


BQ = 1024
_N_BLOCKS = N_TOTAL // BQ


def _topk_body(qt_ref, cf_ref, sh_ref, v_ref, idx_ref, valt_ref):
    s = sh_ref[...]
    qx = qt_ref[0:1, :] - s[0:1, 0:1]
    qy = qt_ref[1:2, :] - s[0:1, 1:2]
    cf = cf_ref[...]
    dx = qx - cf[:, 0:1]
    dy = qy - cf[:, 1:2]
    d = dx * dx + dy * dy
    jj = lax.broadcasted_iota(jnp.int32, (N_IN, BQ), 0).astype(jnp.float32)
    rows = []
    for k in range(K):
        m = jnp.min(d, axis=0, keepdims=True)
        cand = jnp.where(d == m, jj, jnp.float32(N_IN))
        amin = jnp.min(cand, axis=0, keepdims=True)
        rows.append(amin)
        if k + 1 < K:
            d = jnp.where(jj == amin, jnp.float32(jnp.inf), d)
    idx_ref[0:K, :] = jnp.concatenate(rows, axis=0).astype(jnp.int32)
    valt_ref[...] = v_ref[...].T


_topk = pl.pallas_call(
    _topk_body,
    grid=(_N_BLOCKS,),
    in_specs=[
        pl.BlockSpec((2, BQ), lambda i: (0, i)),
        pl.BlockSpec((N_IN, 2), lambda i: (0, 0)),
        pl.BlockSpec((1, 2), lambda i: (0, 0)),
        pl.BlockSpec((C, N_IN // _N_BLOCKS), lambda i: (0, i)),
    ],
    out_specs=(
        pl.BlockSpec((2 * K, BQ), lambda i: (0, i)),
        pl.BlockSpec((N_IN // _N_BLOCKS, C), lambda i: (i, 0)),
    ),
    out_shape=(
        jax.ShapeDtypeStruct((2 * K, N_TOTAL), jnp.int32),
        jax.ShapeDtypeStruct((N_IN, C), jnp.float32),
    ),
)


_NC = 2
_NS = 16
_NW = _NC * _NS
_QPW = N_TOTAL // _NW
_GT = 128
_NT = _QPW // _GT


@functools.cache
def _make_sc_gather_mean():
    @functools.partial(
        pl.kernel,
        mesh=plsc.VectorSubcoreMesh(core_axis_name="c", subcore_axis_name="s"),
        out_type=jax.ShapeDtypeStruct((N_TOTAL, C), jnp.float32),
        scratch_types=[
            pltpu.VMEM((K, _QPW), jnp.int32),
            pltpu.VMEM((_GT, C), jnp.float32),
            pltpu.VMEM((_GT, C), jnp.float32),
            pltpu.VMEM((_GT, C), jnp.float32),
            pltpu.VMEM((_GT, C), jnp.float32),
            pltpu.VMEM((_GT, C), jnp.float32),
            pltpu.VMEM((_GT, C), jnp.float32),
            pltpu.VMEM((_GT, C), jnp.float32),
            pltpu.SemaphoreType.DMA,
            pltpu.SemaphoreType.DMA,
        ],
    )
    def _sc_gather_mean(valt_hbm, idxk_hbm, out_hbm, idx_v, b0, b1, b2, b3,
                        p0, p1, out_v, semA, semB):
        wid = lax.axis_index("s") * _NC + lax.axis_index("c")
        base_q = wid * _QPW

        for k in range(K):
            pltpu.sync_copy(idxk_hbm.at[k, pl.ds(base_q, _QPW)], idx_v.at[k])

        tile0 = (b0, b1, b2, b3)
        tile1 = (p0, p1, b2, b3)
        pendA = [pltpu.async_copy(
            valt_hbm.at[idx_v.at[k, pl.ds(0, _GT)]], tile0[k], semA)
            for k in range(K)]
        pendB = [pltpu.async_copy(
            valt_hbm.at[idx_v.at[k, pl.ds(_GT, _GT)]], tile1[k], semB)
            for k in range(2)]

        def combine(bufs, q_off):
            g0, g1, g2, g3 = bufs

            def one_q(q, carry):
                for c in range(C // 16):
                    sl = pl.ds(c * 16, 16)
                    acc = g0[q, sl] + g1[q, sl]
                    acc = acc + g2[q, sl]
                    acc = acc + g3[q, sl]
                    out_v[q, sl] = acc * 0.25
                return carry

            lax.fori_loop(0, _GT, one_q, 0)
            pltpu.sync_copy(out_v, out_hbm.at[pl.ds(base_q + q_off, _GT)])

        for cp in pendA:
            cp.wait()
        combine(tile0, 0)
        pendB += [pltpu.async_copy(
            valt_hbm.at[idx_v.at[k, pl.ds(_GT, _GT)]], tile1[k], semB)
            for k in range(2, K)]
        for cp in pendB:
            cp.wait()
        combine(tile1, _GT)

    return _sc_gather_mean


_BT = 2048


def _tr_body(x_ref, y_ref):
    y_ref[...] = x_ref[...].T


_transpose = pl.pallas_call(
    _tr_body,
    grid=(N_TOTAL // _BT,),
    in_specs=[pl.BlockSpec((_BT, C), lambda i: (i, 0))],
    out_specs=pl.BlockSpec((C, _BT), lambda i: (0, i)),
    out_shape=jax.ShapeDtypeStruct((C, N_TOTAL), jnp.float32),
)


def kernel(values, coords, new_coords, shift):
    q_t = jnp.concatenate([coords.T, new_coords.T], axis=1)
    idxk, valt = _topk(q_t, coords, shift.reshape(1, 2), values)
    out_t = _make_sc_gather_mean()(valt, idxk)
    return _transpose(out_t)


# speedup vs baseline: 22.8155x; 1.0591x over previous
import functools

import jax
import jax.numpy as jnp
from jax import lax
from jax.experimental import pallas as pl
from jax.experimental.pallas import tpu as pltpu
from jax.experimental.pallas import tpu_sc as plsc

N_IN = 2048
N_TOTAL = 8192
C = 128
K = 4


BQ = 1024
N_HALF = N_TOTAL // 2
_NB_HALF = N_HALF // BQ


def _topk_body(qt_ref, cf_ref, sh_ref, idx_ref):
    s = sh_ref[...]
    qx = qt_ref[0:1, :] - s[0:1, 0:1]
    qy = qt_ref[1:2, :] - s[0:1, 1:2]
    cf = cf_ref[...]
    dx = qx - cf[:, 0:1]
    dy = qy - cf[:, 1:2]
    d = dx * dx + dy * dy
    jj = lax.broadcasted_iota(jnp.int32, (N_IN, BQ), 0).astype(jnp.float32)
    rows = []
    for k in range(K):
        m = jnp.min(d, axis=0, keepdims=True)
        cand = jnp.where(d == m, jj, jnp.float32(N_IN))
        amin = jnp.min(cand, axis=0, keepdims=True)
        rows.append(amin)
        if k + 1 < K:
            d = jnp.where(jj == amin, jnp.float32(jnp.inf), d)
    idx_ref[0:K, :] = jnp.concatenate(rows, axis=0).astype(jnp.int32)


def _topk_valt_body(qt_ref, cf_ref, sh_ref, v_ref, idx_ref, valt_ref):
    _topk_body(qt_ref, cf_ref, sh_ref, idx_ref)
    valt_ref[...] = v_ref[...].T


def _make_topk(half, with_valt):
    qt_spec = pl.BlockSpec((2, BQ), lambda i: (0, i + half * _NB_HALF))
    common = [
        pl.BlockSpec((N_IN, 2), lambda i: (0, 0)),
        pl.BlockSpec((1, 2), lambda i: (0, 0)),
    ]
    idx_spec = pl.BlockSpec((2 * K, BQ), lambda i: (0, i))
    idx_shape = jax.ShapeDtypeStruct((2 * K, N_HALF), jnp.int32)
    if with_valt:
        return pl.pallas_call(
            _topk_valt_body,
            grid=(_NB_HALF,),
            in_specs=[qt_spec] + common + [
                pl.BlockSpec((C, N_IN // _NB_HALF), lambda i: (0, i))],
            out_specs=(idx_spec,
                       pl.BlockSpec((N_IN // _NB_HALF, C), lambda i: (i, 0))),
            out_shape=(idx_shape,
                       jax.ShapeDtypeStruct((N_IN, C), jnp.float32)),
        )
    return pl.pallas_call(
        _topk_body,
        grid=(_NB_HALF,),
        in_specs=[qt_spec] + common,
        out_specs=idx_spec,
        out_shape=idx_shape,
    )


_topk_a = _make_topk(0, True)
_topk_b = _make_topk(1, False)


_NC = 2
_NS = 16
_NW = _NC * _NS
_QPW = N_HALF // _NW


@functools.cache
def _make_sc_gather_mean():
    @functools.partial(
        pl.kernel,
        mesh=plsc.VectorSubcoreMesh(core_axis_name="c", subcore_axis_name="s"),
        out_type=jax.ShapeDtypeStruct((N_HALF, C), jnp.float32),
        scratch_types=[
            pltpu.VMEM((K, _QPW), jnp.int32),
            pltpu.VMEM((_QPW, C), jnp.float32),
            pltpu.VMEM((_QPW, C), jnp.float32),
            pltpu.VMEM((_QPW, C), jnp.float32),
            pltpu.VMEM((_QPW, C), jnp.float32),
            pltpu.VMEM((_QPW, C), jnp.float32),
            pltpu.SemaphoreType.DMA,
            pltpu.SemaphoreType.DMA,
        ],
    )
    def _sc_gather_mean(valt_hbm, idxk_hbm, out_hbm, idx_v, g0, g1, g2, g3,
                        out_v, semI, semG):
        wid = lax.axis_index("s") * _NC + lax.axis_index("c")
        base_q = wid * _QPW

        pendI = [pltpu.async_copy(
            idxk_hbm.at[k, pl.ds(base_q, _QPW)], idx_v.at[k], semI)
            for k in range(K)]
        bufs = (g0, g1, g2, g3)
        pendG = []
        for k in range(K):
            pendI[k].wait()
            pendG.append(pltpu.async_copy(
                valt_hbm.at[idx_v.at[k]], bufs[k], semG))
        for cp in pendG:
            cp.wait()

        def one_q(q, carry):
            for c in range(C // 16):
                sl = pl.ds(c * 16, 16)
                acc = g0[q, sl] + g1[q, sl]
                acc = acc + g2[q, sl]
                acc = acc + g3[q, sl]
                out_v[q, sl] = acc * 0.25
            return carry

        lax.fori_loop(0, _QPW, one_q, 0)
        pltpu.sync_copy(out_v, out_hbm.at[pl.ds(base_q, _QPW)])

    return _sc_gather_mean


_BT = 2048
_NBT = N_TOTAL // _BT


def _tr_body(a_ref, b_ref, y_ref):
    i = pl.program_id(0)
    y_ref[...] = jnp.where(i < _NBT // 2, a_ref[...], b_ref[...]).T


_transpose = pl.pallas_call(
    _tr_body,
    grid=(_NBT,),
    in_specs=[
        pl.BlockSpec((_BT, C), lambda i: (jnp.minimum(i, _NBT // 2 - 1), 0)),
        pl.BlockSpec((_BT, C), lambda i: (jnp.maximum(i - _NBT // 2, 0), 0)),
    ],
    out_specs=pl.BlockSpec((C, _BT), lambda i: (0, i)),
    out_shape=jax.ShapeDtypeStruct((C, N_TOTAL), jnp.float32),
)


def kernel(values, coords, new_coords, shift):
    q_t = jnp.concatenate([coords.T, new_coords.T], axis=1)
    sh = shift.reshape(1, 2)
    sc_gather = _make_sc_gather_mean()
    idxk_a, valt = _topk_a(q_t, coords, sh, values)
    out_a = sc_gather(valt, idxk_a)
    idxk_b = _topk_b(q_t, coords, sh)
    out_b = sc_gather(valt, idxk_b)
    return _transpose(out_a, out_b)
